# Initial kernel scaffold; baseline (speedup 1.0000x reference)
import jax
import jax.numpy as jnp
from jax.experimental import pallas as pl


def kernel(relation_raw_embed, rule_matrix, batch_query_rels, nnz_batch_ids, nnz_ent_ids, nnz_rule_ids, nnz_time_delta, rel_proj_W, rel_proj_b, lstm_W_ih, lstm_W_hh, lstm_b, relmlp_W1, relmlp_b1, relmlp_W2, relmlp_b2):
    raise NotImplementedError("write your pallas kernel here")


# speedup vs baseline: 52.8380x; 52.8380x over previous
import jax
import jax.numpy as jnp
from jax import lax
from jax.experimental import pallas as pl
from jax.experimental.pallas import tpu as pltpu
from jax.experimental.pallas import tpu_sc as plsc

ENT_NUM = 50000
RULE_NUM = 10000
REL_NUM = 500
D = 128
B = 16
NNZ = 1600000
REL_PAD = 512
ENT_PAD = 50176
NW = 32
CHUNK = 2048
RBLK = 1000


def _proj_body(raw_ref, w_ref, b_ref, q_ref, w1_ref, b1_ref, w2_ref, b2_ref,
               e_ref, btw_ref):
    e = jnp.dot(raw_ref[...], w_ref[...], preferred_element_type=jnp.float32)
    e = e + b_ref[...]
    row = lax.broadcasted_iota(jnp.int32, (REL_PAD, 1), 0)
    e = jnp.where(row < REL_NUM, e, 0.0)
    e_ref[...] = e
    qoh = (lax.broadcasted_iota(jnp.int32, (B, REL_PAD), 1) == q_ref[...])
    qe = jnp.dot(qoh.astype(jnp.float32), e, preferred_element_type=jnp.float32)
    h = jnp.maximum(
        jnp.dot(qe, w1_ref[...], preferred_element_type=jnp.float32) + b1_ref[...],
        0.0)
    z = jnp.dot(h, w2_ref[...], preferred_element_type=jnp.float32) + b2_ref[...]
    btw_ref[...] = jax.nn.sigmoid(z)


def _rule_body(rm_ref, e_ref, wih_ref, whh_ref, lb_ref, out_ref):
    ids = rm_ref[...]
    e = e_ref[...]
    col = lax.broadcasted_iota(jnp.int32, (RBLK, REL_PAD), 1)

    def emb(t):
        oh = (col == ids[:, t:t + 1]).astype(jnp.float32)
        return jnp.dot(oh, e, preferred_element_type=jnp.float32)

    qe = emb(0)
    wih = wih_ref[...]
    whh = whh_ref[...]
    lb = lb_ref[...]
    h = None
    c = None
    for t in (1, 2, 3):
        x = emb(t)
        g = jnp.dot(x, wih, preferred_element_type=jnp.float32) + lb
        if h is not None:
            g = g + jnp.dot(h, whh, preferred_element_type=jnp.float32)
        i = jax.nn.sigmoid(g[:, 0:D])
        f = jax.nn.sigmoid(g[:, D:2 * D])
        gg = jnp.tanh(g[:, 2 * D:3 * D])
        o = jax.nn.sigmoid(g[:, 3 * D:4 * D])
        c = f * c + i * gg if c is not None else i * gg
        h = o * jnp.tanh(c)
    eps = 1e-8
    qn = jnp.maximum(jnp.sqrt(jnp.sum(qe * qe, axis=1, keepdims=True)), eps)
    bn = jnp.maximum(jnp.sqrt(jnp.sum(h * h, axis=1, keepdims=True)), eps)
    out_ref[...] = jnp.sum(qe * h, axis=1, keepdims=True) / (qn * bn)


def _sc_body(ent_ref, rul_ref, td_ref, rtab_ref, ma_ref,
             pt_ref, pe_ref,
             ent_v, rul_v, td_v, rtab_v, meta_v, acc_t, acc_e):
    wid = lax.axis_index("s") * 2 + lax.axis_index("c")

    zeros = jnp.zeros((16,), jnp.float32)

    def zbody(i, carry):
        acc_t[pl.ds(i * 16, 16)] = zeros
        acc_e[pl.ds(i * 16, 16)] = zeros
        return carry

    lax.fori_loop(0, ENT_PAD // 16, zbody, 0)

    pltpu.sync_copy(rtab_ref, rtab_v)

    pltpu.sync_copy(ma_ref.at[wid], meta_v)
    mv = meta_v[pl.ds(0, 16)]
    astart = mv[0]
    nchunks = mv[1]
    start_vec = jnp.full((16,), mv[2], jnp.int32)
    end_vec = jnp.full((16,), mv[3], jnp.int32)
    lane = lax.iota(jnp.int32, 16)

    def chunk_body(ci, carry):
        chunk_lo = astart + ci * CHUNK
        off = pl.multiple_of(jnp.minimum(chunk_lo, NNZ - CHUNK), 16)
        pltpu.sync_copy(ent_ref.at[pl.ds(off, CHUNK)], ent_v)
        pltpu.sync_copy(rul_ref.at[pl.ds(off, CHUNK)], rul_v)
        pltpu.sync_copy(td_ref.at[pl.ds(off, CHUNK)], td_v)
        lo_vec = jnp.maximum(start_vec, jnp.full((16,), chunk_lo, jnp.int32))
        g0 = jnp.full((16,), off, jnp.int32) + lane

        def inner(j, g):
            e = ent_v[pl.ds(j * 16, 16)]
            r = rul_v[pl.ds(j * 16, 16)]
            t = td_v[pl.ds(j * 16, 16)]
            m = (g >= lo_vec) & (g < end_vec)
            tv = jnp.exp(t * -0.1)
            plsc.addupdate_scatter(acc_t, [e], tv, mask=m)
            rv = plsc.load_gather(rtab_v, [r], mask=m)
            plsc.addupdate_scatter(acc_e, [e], rv, mask=m)
            return g + 16

        lax.fori_loop(0, CHUNK // 16, inner, g0)
        return carry

    lax.fori_loop(0, nchunks, chunk_body, 0)

    pltpu.sync_copy(acc_t, pt_ref.at[wid])
    pltpu.sync_copy(acc_e, pe_ref.at[wid])


def _combine_body(pt_ref, pe_ref, btw_ref, out_ref):
    pt = pt_ref[...]
    pe = pe_ref[...]
    t = pt[:, 0, :] + pt[:, 1, :]
    e = pe[:, 0, :] + pe[:, 1, :]
    col = lax.broadcasted_iota(jnp.int32, (1, ENT_PAD), 1)
    valid = col < ENT_NUM

    def sm(x):
        xm = jnp.where(valid, x, -1e30)
        m = jnp.max(xm, axis=1, keepdims=True)
        ex = jnp.where(valid, jnp.exp(x - m), 0.0)
        s = jnp.sum(ex, axis=1, keepdims=True)
        return ex / s

    ts = sm(t)
    es = sm(e)
    w = btw_ref[...].reshape(1, 1)
    res = (1.0 - w) * es + w * ts
    out_ref[...] = res[:, :ENT_NUM].reshape(1, 1, ENT_NUM)


def kernel(relation_raw_embed, rule_matrix, batch_query_rels, nnz_batch_ids,
           nnz_ent_ids, nnz_rule_ids, nnz_time_delta, rel_proj_W, rel_proj_b,
           lstm_W_ih, lstm_W_hh, lstm_b, relmlp_W1, relmlp_b1, relmlp_W2,
           relmlp_b2):
    f32 = jnp.float32
    raw = jnp.pad(relation_raw_embed.astype(f32),
                  ((0, REL_PAD - REL_NUM), (0, 0)))
    q = batch_query_rels.astype(jnp.int32).reshape(B, 1)
    rm = rule_matrix.astype(jnp.int32)
    bids = nnz_batch_ids.astype(jnp.int32)
    ent = nnz_ent_ids.astype(jnp.int32)
    rul = nnz_rule_ids.astype(jnp.int32)
    td = nnz_time_delta.astype(f32)

    e_emb, btw = pl.pallas_call(
        _proj_body,
        out_shape=[jax.ShapeDtypeStruct((REL_PAD, D), f32),
                   jax.ShapeDtypeStruct((B, 1), f32)],
    )(raw, rel_proj_W.astype(f32), rel_proj_b.astype(f32).reshape(1, D), q,
      relmlp_W1.astype(f32), relmlp_b1.astype(f32).reshape(1, D),
      relmlp_W2.astype(f32), relmlp_b2.astype(f32).reshape(1, 1))

    rs = pl.pallas_call(
        _rule_body,
        grid=(RULE_NUM // RBLK,),
        in_specs=[pl.BlockSpec((RBLK, 4), lambda i: (i, 0)),
                  pl.BlockSpec((REL_PAD, D), lambda i: (0, 0)),
                  pl.BlockSpec((D, 4 * D), lambda i: (0, 0)),
                  pl.BlockSpec((D, 4 * D), lambda i: (0, 0)),
                  pl.BlockSpec((1, 4 * D), lambda i: (0, 0))],
        out_specs=pl.BlockSpec((RBLK, 1), lambda i: (i, 0)),
        out_shape=jax.ShapeDtypeStruct((RULE_NUM, 1), f32),
    )(rm, e_emb, lstm_W_ih.astype(f32), lstm_W_hh.astype(f32),
      lstm_b.astype(f32).reshape(1, 4 * D))

    barange = jnp.arange(B, dtype=bids.dtype)
    starts = jnp.searchsorted(bids, barange, side="left").astype(jnp.int32)
    ends = jnp.searchsorted(bids, barange, side="right").astype(jnp.int32)
    w = jnp.arange(NW, dtype=jnp.int32)
    bb = w // 2
    hh = w % 2
    lo = starts[bb]
    hi = ends[bb]
    mid = lo + (hi - lo) // 2
    tile_start = jnp.where(hh == 0, lo, mid)
    tile_end = jnp.where(hh == 0, mid, hi)
    astart = (tile_start // 16) * 16
    nch = (tile_end - astart + CHUNK - 1) // CHUNK
    meta = jnp.stack([astart, nch, tile_start, tile_end], axis=1)
    meta = jnp.pad(meta.astype(jnp.int32), ((0, 0), (0, 128 - 4)))

    sc_fn = pl.kernel(
        _sc_body,
        out_type=[jax.ShapeDtypeStruct((NW, ENT_PAD), f32),
                  jax.ShapeDtypeStruct((NW, ENT_PAD), f32)],
        mesh=plsc.VectorSubcoreMesh(core_axis_name="c", subcore_axis_name="s"),
        compiler_params=pltpu.CompilerParams(needs_layout_passes=False),
        scratch_types=[pltpu.VMEM((CHUNK,), jnp.int32),
                       pltpu.VMEM((CHUNK,), jnp.int32),
                       pltpu.VMEM((CHUNK,), f32),
                       pltpu.VMEM((RULE_NUM,), f32),
                       pltpu.VMEM((128,), jnp.int32),
                       pltpu.VMEM((ENT_PAD,), f32),
                       pltpu.VMEM((ENT_PAD,), f32)],
    )
    pt, pe = sc_fn(ent, rul, td, rs.reshape(RULE_NUM), meta)

    out = pl.pallas_call(
        _combine_body,
        grid=(B,),
        in_specs=[pl.BlockSpec((1, 2, ENT_PAD), lambda b: (b, 0, 0)),
                  pl.BlockSpec((1, 2, ENT_PAD), lambda b: (b, 0, 0)),
                  pl.BlockSpec((1, 1, 1), lambda b: (b, 0, 0))],
        out_specs=pl.BlockSpec((1, 1, ENT_NUM), lambda b: (b, 0, 0)),
        out_shape=jax.ShapeDtypeStruct((B, 1, ENT_NUM), f32),
    )(pt.reshape(B, 2, ENT_PAD), pe.reshape(B, 2, ENT_PAD),
      btw.reshape(B, 1, 1))
    return out.reshape(B, ENT_NUM), rs
